# baseline (speedup 1.0000x reference)
import jax
import jax.numpy as jnp
from jax import lax
from jax.experimental import pallas as pl
from jax.experimental.pallas import tpu as pltpu
from jax.experimental.pallas import tpu_sc as plsc

N = 10000
NP = 10240
E = 320000
B = 128
NC = 2
NS = 16
EP = 327680
NBB = EP // B
RT = NP // NS
RB = 1024
SLAB = 8


def _make_spmm(tail, dtype, untiled, nbt0, nbt1):
    row_shape = (B,) + tail
    acc_shape = (NP,) + tail

    def body(tab, zeros_hbm, srcs, dsts, out, idxs_v, idxd_v, rows0, rows1,
             acc_sh, gsem0, gsem1, ssem0, ssem1):
        c = lax.axis_index("c")
        s = lax.axis_index("s")
        rows = [rows0, rows1]
        gsem = [gsem0, gsem1]
        ssem = [ssem0, ssem1]

        for k in range(RT // B):
            pltpu.sync_copy(zeros_hbm, acc_sh.at[pl.ds(s * RT + k * B, B)])
        plsc.subcore_barrier()

        def edge_loop(nbt_c, base):
            boff = base + s * nbt_c

            def slab_body(si, carry):
                sboff = boff + si * SLAB
                pltpu.sync_copy(dsts.at[pl.ds(sboff, SLAB)], idxd_v)
                pltpu.sync_copy(srcs.at[pl.ds(sboff, SLAB)], idxs_v)
                gd = {0: pltpu.async_copy(tab.at[idxs_v.at[0]], rows[0],
                                          gsem[0])}
                sd = {}
                for b in range(SLAB):
                    cur = b % 2
                    nxt = (b + 1) % 2
                    gd[b].wait()
                    if b + 1 < SLAB:
                        if b - 1 in sd:
                            sd[b - 1].wait()
                        gd[b + 1] = pltpu.async_copy(
                            tab.at[idxs_v.at[b + 1]], rows[nxt], gsem[nxt])
                    sd[b] = pltpu.async_copy(
                        rows[cur], acc_sh.at[idxd_v.at[b]], ssem[cur],
                        add=True)
                sd[SLAB - 1].wait()
                sd[SLAB - 2].wait()
                return carry

            lax.fori_loop(0, nbt_c // SLAB, slab_body, 0)

        if nbt0 > 0:
            @pl.when(c == 0)
            def _():
                edge_loop(nbt0, 0)
        if nbt1 > 0:
            @pl.when(c == 1)
            def _():
                edge_loop(nbt1, nbt0 * NS)

        plsc.subcore_barrier()
        pltpu.sync_copy(acc_sh.at[pl.ds(s * RT, RT)],
                        out.at[c, pl.ds(s * RT, RT)])

    cp = pltpu.CompilerParams(use_tc_tiling_on_sc=False) if untiled else None
    return pl.kernel(
        body,
        out_type=jax.ShapeDtypeStruct((NC,) + acc_shape, dtype),
        mesh=plsc.VectorSubcoreMesh(core_axis_name="c", subcore_axis_name="s"),
        scratch_types=[
            pltpu.VMEM((SLAB, B), jnp.int32),
            pltpu.VMEM((SLAB, B), jnp.int32),
            pltpu.VMEM(row_shape, dtype),
            pltpu.VMEM(row_shape, dtype),
            pltpu.VMEM_SHARED(acc_shape, dtype),
            pltpu.SemaphoreType.DMA,
            pltpu.SemaphoreType.DMA,
            pltpu.SemaphoreType.DMA,
            pltpu.SemaphoreType.DMA,
        ],
        compiler_params=cp,
    )


def _make_deg_pass():
    F = 16
    nbt = NBB // (NC * NS)
    nslab = nbt // SLAB

    def _fill_zero(ref):
        z16 = jnp.zeros((16,), jnp.float32)

        def zb(i, _):
            r = i // (F // 16)
            j = i % (F // 16)
            ref[r, pl.ds(j * 16, 16)] = z16
            return 0
        lax.fori_loop(0, B * (F // 16), zb, 0)

    def body(ones_hbm, dsts, out, idxd_v, rows_v, acc_sh):
        c = lax.axis_index("c")
        s = lax.axis_index("s")
        boff = c * (nbt * NS) + s * nbt
        _fill_zero(rows_v)
        for k in range(RT // B):
            pltpu.sync_copy(rows_v, acc_sh.at[pl.ds(s * RT + k * B, B)])
        pltpu.sync_copy(ones_hbm, rows_v)
        plsc.subcore_barrier()

        def slab_body(si, carry):
            pltpu.sync_copy(dsts.at[pl.ds(boff + si * SLAB, SLAB)], idxd_v)

            def step(b, inner):
                pltpu.sync_copy(rows_v, acc_sh.at[idxd_v.at[b]], add=True)
                return inner
            lax.fori_loop(0, SLAB, step, 0)
            return carry
        lax.fori_loop(0, nslab, slab_body, 0)

        plsc.subcore_barrier()
        pltpu.sync_copy(acc_sh.at[pl.ds(s * RT, RT)],
                        out.at[c, pl.ds(s * RT, RT)])

    return pl.kernel(
        body,
        out_type=jax.ShapeDtypeStruct((NC, NP, F), jnp.float32),
        mesh=plsc.VectorSubcoreMesh(core_axis_name="c", subcore_axis_name="s"),
        scratch_types=[
            pltpu.VMEM((SLAB, B), jnp.int32),
            pltpu.VMEM((B, F), jnp.float32),
            pltpu.VMEM_SHARED((NP, F), jnp.float32),
        ],
        compiler_params=pltpu.CompilerParams(use_tc_tiling_on_sc=False),
    )


def _dinv_of(d_ref):
    deg = d_ref[0] + d_ref[1] + 1.0
    return lax.rsqrt(jnp.maximum(deg, 1.0))


def _tc1_body(x_ref, w_ref, d_ref, o_ref):
    dinv = _dinv_of(d_ref)
    h = jnp.dot(x_ref[...], w_ref[...], preferred_element_type=jnp.float32)
    o_ref[...] = (h * dinv[:, None]).astype(jnp.bfloat16)


def _tc2_body(a_ref, g_ref, d_ref, b_ref, w_ref, o_ref):
    dinv = _dinv_of(d_ref)
    h2 = None
    for cc in range(2):
        acc_c = (a_ref[0, :, cc, :].astype(jnp.float32)
                 + a_ref[1, :, cc, :].astype(jnp.float32))
        g_c = g_ref[:, cc * 128:(cc + 1) * 128].astype(jnp.float32)
        r = (acc_c + g_c) * dinv[:, None] + b_ref[cc][None, :]
        r = jnp.maximum(r, 0.0)
        p = jnp.dot(r, w_ref[cc], preferred_element_type=jnp.float32)
        h2 = p if h2 is None else h2 + p
    o_ref[...] = h2 * dinv[:, None]


def _tc3_body(a_ref, g_ref, d_ref, b_ref, o_ref):
    dinv = _dinv_of(d_ref)
    o_ref[...] = ((a_ref[0] + a_ref[1] + g_ref[...]) * dinv[:, None]
                  + b_ref[0][None, :])


NBT0 = 152
NBT1 = 8


@jax.jit
def kernel(x, edge_index, W1, b1, W2, b2):
    src = edge_index[0]
    dst = edge_index[1]
    padn = EP - E
    fill = jnp.full((padn,), N, jnp.int32)
    src2d = jnp.concatenate([src, fill]).reshape(NBB, B)
    dst2d = jnp.concatenate([dst, fill]).reshape(NBB, B)
    xp = jnp.zeros((NP, 128), jnp.float32).at[:N].set(x)
    b1r = b1.reshape(2, 128)
    w2r = W2.reshape(2, 128, 128)
    b2r = b2.reshape(1, 128)

    ones_hbm = jnp.ones((B, 16), jnp.float32)
    degp = _make_deg_pass()(ones_hbm, dst2d)
    degcol = degp[:, :, 0]

    g1 = pl.pallas_call(
        _tc1_body,
        grid=(NP // RB,),
        in_specs=[
            pl.BlockSpec((RB, 128), lambda r: (r, 0)),
            pl.BlockSpec((128, 256), lambda r: (0, 0)),
            pl.BlockSpec((2, RB), lambda r: (0, r)),
        ],
        out_specs=pl.BlockSpec((RB, 256), lambda r: (r, 0)),
        out_shape=jax.ShapeDtypeStruct((NP, 256), jnp.bfloat16),
    )(xp, W1, degcol)

    tab1 = g1.reshape(NP, 2, 128)
    zeros_bf = jnp.zeros((B, 2, 128), jnp.bfloat16)
    acc1 = _make_spmm((2, 128), jnp.bfloat16, True, NBT0, NBT1)(
        tab1, zeros_bf, src2d, dst2d)

    g2 = pl.pallas_call(
        _tc2_body,
        grid=(NP // RB,),
        in_specs=[
            pl.BlockSpec((2, RB, 2, 128), lambda r: (0, r, 0, 0)),
            pl.BlockSpec((RB, 256), lambda r: (r, 0)),
            pl.BlockSpec((2, RB), lambda r: (0, r)),
            pl.BlockSpec((2, 128), lambda r: (0, 0)),
            pl.BlockSpec((2, 128, 128), lambda r: (0, 0, 0)),
        ],
        out_specs=pl.BlockSpec((RB, 128), lambda r: (r, 0)),
        out_shape=jax.ShapeDtypeStruct((NP, 128), jnp.float32),
    )(acc1, g1, degcol, b1r, w2r)

    zeros_f32 = jnp.zeros((B, 128), jnp.float32)
    acc2 = _make_spmm((128,), jnp.float32, False, NBT0, NBT1)(
        g2, zeros_f32, src2d, dst2d)

    z = pl.pallas_call(
        _tc3_body,
        grid=(NP // RB,),
        in_specs=[
            pl.BlockSpec((2, RB, 128), lambda r: (0, r, 0)),
            pl.BlockSpec((RB, 128), lambda r: (r, 0)),
            pl.BlockSpec((2, RB), lambda r: (0, r)),
            pl.BlockSpec((1, 128), lambda r: (0, 0)),
        ],
        out_specs=pl.BlockSpec((RB, 128), lambda r: (r, 0)),
        out_shape=jax.ShapeDtypeStruct((NP, 128), jnp.float32),
    )(acc2, g2, degcol, b2r)

    return z[:N]


# speedup vs baseline: 1.1484x; 1.1484x over previous
import jax
import jax.numpy as jnp
from jax import lax
from jax.experimental import pallas as pl
from jax.experimental.pallas import tpu as pltpu
from jax.experimental.pallas import tpu_sc as plsc

N = 10000
NP = 10240
E = 320000
B = 128
NC = 2
NS = 16
EP = 327680
NBB = EP // B
RT = NP // NS
RB = 1024
SLAB = 16


def _make_spmm(tail, dtype, untiled, nbt0, nbt1):
    row_shape = (B,) + tail
    acc_shape = (NP,) + tail

    def body(tab, zeros_hbm, srcs, dsts, out, idxs_v, idxd_v, rows0, rows1,
             acc_sh, gsem0, gsem1, ssem0, ssem1):
        c = lax.axis_index("c")
        s = lax.axis_index("s")
        rows = [rows0, rows1]
        gsem = [gsem0, gsem1]
        ssem = [ssem0, ssem1]

        for k in range(RT // B):
            pltpu.sync_copy(zeros_hbm, acc_sh.at[pl.ds(s * RT + k * B, B)])
        plsc.subcore_barrier()

        def edge_loop(nbt_c, base):
            boff = base + s * nbt_c

            def slab_body(si, carry):
                sboff = boff + si * SLAB
                pltpu.sync_copy(dsts.at[pl.ds(sboff, SLAB)], idxd_v)
                pltpu.sync_copy(srcs.at[pl.ds(sboff, SLAB)], idxs_v)
                gd = {0: pltpu.async_copy(tab.at[idxs_v.at[0]], rows[0],
                                          gsem[0])}
                sd = {}
                for b in range(SLAB):
                    cur = b % 2
                    nxt = (b + 1) % 2
                    gd[b].wait()
                    if b + 1 < SLAB:
                        if b - 1 in sd:
                            sd[b - 1].wait()
                        gd[b + 1] = pltpu.async_copy(
                            tab.at[idxs_v.at[b + 1]], rows[nxt], gsem[nxt])
                    sd[b] = pltpu.async_copy(
                        rows[cur], acc_sh.at[idxd_v.at[b]], ssem[cur],
                        add=True)
                sd[SLAB - 1].wait()
                sd[SLAB - 2].wait()
                return carry

            lax.fori_loop(0, nbt_c // SLAB, slab_body, 0)

        if nbt0 > 0:
            @pl.when(c == 0)
            def _():
                edge_loop(nbt0, 0)
        if nbt1 > 0:
            @pl.when(c == 1)
            def _():
                edge_loop(nbt1, nbt0 * NS)

        plsc.subcore_barrier()
        pltpu.sync_copy(acc_sh.at[pl.ds(s * RT, RT)],
                        out.at[c, pl.ds(s * RT, RT)])

    cp = pltpu.CompilerParams(use_tc_tiling_on_sc=False) if untiled else None
    return pl.kernel(
        body,
        out_type=jax.ShapeDtypeStruct((NC,) + acc_shape, dtype),
        mesh=plsc.VectorSubcoreMesh(core_axis_name="c", subcore_axis_name="s"),
        scratch_types=[
            pltpu.VMEM((SLAB, B), jnp.int32),
            pltpu.VMEM((SLAB, B), jnp.int32),
            pltpu.VMEM(row_shape, dtype),
            pltpu.VMEM(row_shape, dtype),
            pltpu.VMEM_SHARED(acc_shape, dtype),
            pltpu.SemaphoreType.DMA,
            pltpu.SemaphoreType.DMA,
            pltpu.SemaphoreType.DMA,
            pltpu.SemaphoreType.DMA,
        ],
        compiler_params=cp,
    )


def _make_deg_pass():
    F = 16
    nbt = NBB // (NC * NS)
    nslab = nbt // SLAB

    def _fill_zero(ref):
        z16 = jnp.zeros((16,), jnp.float32)

        def zb(i, _):
            r = i // (F // 16)
            j = i % (F // 16)
            ref[r, pl.ds(j * 16, 16)] = z16
            return 0
        lax.fori_loop(0, B * (F // 16), zb, 0)

    def body(ones_hbm, dsts, out, idxd_v, rows_v, acc_sh):
        c = lax.axis_index("c")
        s = lax.axis_index("s")
        boff = c * (nbt * NS) + s * nbt
        _fill_zero(rows_v)
        for k in range(RT // B):
            pltpu.sync_copy(rows_v, acc_sh.at[pl.ds(s * RT + k * B, B)])
        pltpu.sync_copy(ones_hbm, rows_v)
        plsc.subcore_barrier()

        def slab_body(si, carry):
            pltpu.sync_copy(dsts.at[pl.ds(boff + si * SLAB, SLAB)], idxd_v)

            def step(b, inner):
                pltpu.sync_copy(rows_v, acc_sh.at[idxd_v.at[b]], add=True)
                return inner
            lax.fori_loop(0, SLAB, step, 0)
            return carry
        lax.fori_loop(0, nslab, slab_body, 0)

        plsc.subcore_barrier()
        pltpu.sync_copy(acc_sh.at[pl.ds(s * RT, RT)],
                        out.at[c, pl.ds(s * RT, RT)])

    return pl.kernel(
        body,
        out_type=jax.ShapeDtypeStruct((NC, NP, F), jnp.float32),
        mesh=plsc.VectorSubcoreMesh(core_axis_name="c", subcore_axis_name="s"),
        scratch_types=[
            pltpu.VMEM((SLAB, B), jnp.int32),
            pltpu.VMEM((B, F), jnp.float32),
            pltpu.VMEM_SHARED((NP, F), jnp.float32),
        ],
        compiler_params=pltpu.CompilerParams(use_tc_tiling_on_sc=False),
    )


def _dinv_of(d_ref):
    deg = d_ref[0] + d_ref[1] + 1.0
    return lax.rsqrt(jnp.maximum(deg, 1.0))


def _tc1_body(x_ref, w_ref, d_ref, o_ref):
    dinv = _dinv_of(d_ref)
    h = jnp.dot(x_ref[...], w_ref[...], preferred_element_type=jnp.float32)
    o_ref[...] = (h * dinv[:, None]).astype(jnp.bfloat16)


def _tc2_body(a_ref, g_ref, d_ref, b_ref, w_ref, o_ref):
    dinv = _dinv_of(d_ref)
    h2 = None
    for cc in range(2):
        acc_c = (a_ref[0, :, cc, :].astype(jnp.float32)
                 + a_ref[1, :, cc, :].astype(jnp.float32))
        g_c = g_ref[:, cc * 128:(cc + 1) * 128].astype(jnp.float32)
        r = (acc_c + g_c) * dinv[:, None] + b_ref[cc][None, :]
        r = jnp.maximum(r, 0.0)
        p = jnp.dot(r, w_ref[cc], preferred_element_type=jnp.float32)
        h2 = p if h2 is None else h2 + p
    o_ref[...] = (h2 * dinv[:, None]).astype(jnp.bfloat16)


def _tc3_body(a_ref, g_ref, d_ref, b_ref, o_ref):
    dinv = _dinv_of(d_ref)
    acc = (a_ref[0, :, 0, :].astype(jnp.float32)
           + a_ref[1, :, 0, :].astype(jnp.float32))
    o_ref[...] = ((acc + g_ref[...].astype(jnp.float32)) * dinv[:, None]
                  + b_ref[0][None, :])


NBT0 = 144
NBT1 = 16


@jax.jit
def kernel(x, edge_index, W1, b1, W2, b2):
    src = edge_index[0]
    dst = edge_index[1]
    padn = EP - E
    fill = jnp.full((padn,), N, jnp.int32)
    src2d = jnp.concatenate([src, fill]).reshape(NBB, B)
    dst2d = jnp.concatenate([dst, fill]).reshape(NBB, B)
    xp = jnp.zeros((NP, 128), jnp.float32).at[:N].set(x)
    b1r = b1.reshape(2, 128)
    w2r = W2.reshape(2, 128, 128)
    b2r = b2.reshape(1, 128)

    ones_hbm = jnp.ones((B, 16), jnp.float32)
    degp = _make_deg_pass()(ones_hbm, dst2d)
    degcol = degp[:, :, 0]

    g1 = pl.pallas_call(
        _tc1_body,
        grid=(NP // RB,),
        in_specs=[
            pl.BlockSpec((RB, 128), lambda r: (r, 0)),
            pl.BlockSpec((128, 256), lambda r: (0, 0)),
            pl.BlockSpec((2, RB), lambda r: (0, r)),
        ],
        out_specs=pl.BlockSpec((RB, 256), lambda r: (r, 0)),
        out_shape=jax.ShapeDtypeStruct((NP, 256), jnp.bfloat16),
    )(xp, W1, degcol)

    tab1 = g1.reshape(NP, 2, 128)
    zeros_bf = jnp.zeros((B, 2, 128), jnp.bfloat16)
    acc1 = _make_spmm((2, 128), jnp.bfloat16, True, NBT0, NBT1)(
        tab1, zeros_bf, src2d, dst2d)

    g2 = pl.pallas_call(
        _tc2_body,
        grid=(NP // RB,),
        in_specs=[
            pl.BlockSpec((2, RB, 2, 128), lambda r: (0, r, 0, 0)),
            pl.BlockSpec((RB, 256), lambda r: (r, 0)),
            pl.BlockSpec((2, RB), lambda r: (0, r)),
            pl.BlockSpec((2, 128), lambda r: (0, 0)),
            pl.BlockSpec((2, 128, 128), lambda r: (0, 0, 0)),
        ],
        out_specs=pl.BlockSpec((RB, 128), lambda r: (r, 0)),
        out_shape=jax.ShapeDtypeStruct((NP, 128), jnp.bfloat16),
    )(acc1, g1, degcol, b1r, w2r)

    tab2 = g2.reshape(NP, 1, 128)
    zeros_bf2 = jnp.zeros((B, 1, 128), jnp.bfloat16)
    acc2 = _make_spmm((1, 128), jnp.bfloat16, True, NBT0, NBT1)(
        tab2, zeros_bf2, src2d, dst2d)

    z = pl.pallas_call(
        _tc3_body,
        grid=(NP // RB,),
        in_specs=[
            pl.BlockSpec((2, RB, 1, 128), lambda r: (0, r, 0, 0)),
            pl.BlockSpec((RB, 128), lambda r: (r, 0)),
            pl.BlockSpec((2, RB), lambda r: (0, r)),
            pl.BlockSpec((1, 128), lambda r: (0, 0)),
        ],
        out_specs=pl.BlockSpec((RB, 128), lambda r: (r, 0)),
        out_shape=jax.ShapeDtypeStruct((NP, 128), jnp.float32),
    )(acc2, g2, degcol, b2r)

    return z[:N]
